# Initial kernel scaffold; baseline (speedup 1.0000x reference)
import jax
import jax.numpy as jnp
from jax.experimental import pallas as pl


def kernel(x, centroids, base_set):
    raise NotImplementedError("write your pallas kernel here")


# speedup vs baseline: 7.6031x; 7.6031x over previous
import functools

import jax
import jax.numpy as jnp
from jax.experimental import pallas as pl
from jax.experimental.pallas import tpu as pltpu

N = 65536
D = 512
M = 400
MP = 512
NBIT = 64
K = 5
BANDWIDTH = 512.0
BLOCK = 1024


def _body(x_ref, ct_ref, bias_ref, bs_ref, y_ref, q_ref, *, nsteps):
    xb = x_ref[...]
    ct = ct_ref[...]
    c_sq = jnp.sum(ct * ct, axis=0, keepdims=True) + bias_ref[...]
    qc = jnp.dot(xb, ct, preferred_element_type=jnp.float32)
    s = c_sq - 2.0 * qc

    iota = jax.lax.broadcasted_iota(jnp.int32, s.shape, 1)
    w_mat = jnp.zeros_like(s)
    val0 = None
    wsum = None
    for k in range(K):
        val = jnp.min(s, axis=1, keepdims=True)
        idx = jnp.argmin(s, axis=1, keepdims=True)
        onehot = iota == idx
        if k == 0:
            val0 = val
            w = jnp.ones_like(val)
            wsum = w
        else:
            w = jnp.exp((val0 - val) * (1.0 / BANDWIDTH))
            wsum = wsum + w
        w_mat = jnp.where(onehot, w, w_mat)
        if k < K - 1:
            s = jnp.where(onehot, jnp.float32(jnp.inf), s)

    y = jnp.dot(w_mat, bs_ref[...], preferred_element_type=jnp.float32)
    y = y / wsum
    y_ref[...] = y

    vs = jnp.sign(y)
    nv = jnp.maximum(jnp.sqrt(jnp.sum(y * y, axis=1, keepdims=True)), 1e-8)
    ns = jnp.maximum(jnp.sqrt(jnp.sum(vs * vs, axis=1, keepdims=True)), 1e-8)
    cos = jnp.sum(y * vs, axis=1, keepdims=True) / (nv * ns)
    blocksum = jnp.sum(1.0 - cos).reshape(1, 1)

    i = pl.program_id(0)

    @pl.when(i == 0)
    def _init():
        q_ref[...] = jnp.zeros_like(q_ref)

    q_ref[...] += blocksum

    @pl.when(i == nsteps - 1)
    def _fin():
        q_ref[...] = q_ref[...] * (1.0 / N)


@jax.jit
def kernel(x, centroids, base_set):
    ct = jnp.zeros((D, MP), jnp.float32).at[:, :M].set(centroids.T)
    bias = jnp.zeros((1, MP), jnp.float32).at[0, M:].set(3e38)
    bs = jnp.zeros((MP, NBIT), jnp.float32).at[:M, :].set(base_set)

    nsteps = N // BLOCK
    y, q = pl.pallas_call(
        functools.partial(_body, nsteps=nsteps),
        grid=(nsteps,),
        in_specs=[
            pl.BlockSpec((BLOCK, D), lambda i: (i, 0)),
            pl.BlockSpec((D, MP), lambda i: (0, 0)),
            pl.BlockSpec((1, MP), lambda i: (0, 0)),
            pl.BlockSpec((MP, NBIT), lambda i: (0, 0)),
        ],
        out_specs=[
            pl.BlockSpec((BLOCK, NBIT), lambda i: (i, 0)),
            pl.BlockSpec((1, 1), lambda i: (0, 0)),
        ],
        out_shape=[
            jax.ShapeDtypeStruct((N, NBIT), jnp.float32),
            jax.ShapeDtypeStruct((1, 1), jnp.float32),
        ],
        compiler_params=pltpu.CompilerParams(
            dimension_semantics=("arbitrary",),
        ),
    )(x, ct, bias, bs)
    return y, q[0, 0]


# speedup vs baseline: 16.3771x; 2.1540x over previous
import functools

import jax
import jax.numpy as jnp
from jax.experimental import pallas as pl
from jax.experimental.pallas import tpu as pltpu

N = 65536
D = 512
M = 400
MP = 512
NBIT = 64
K = 5
BANDWIDTH = 512.0
BLOCK = 1024


def _body(x_ref, ct_ref, bias_ref, bs_ref, y_ref, q_ref, *, nsteps):
    xb = x_ref[...]
    ct = ct_ref[...]
    c_sq = jnp.sum(ct * ct, axis=0, keepdims=True) + bias_ref[...]
    qc = jnp.dot(xb, ct, preferred_element_type=jnp.float32)
    s = c_sq - 2.0 * qc
    st = s.T

    iota = jax.lax.broadcasted_iota(jnp.int32, st.shape, 0)
    w_mat = jnp.zeros_like(st)
    val0 = None
    wsum = None
    for k in range(K):
        val = jnp.min(st, axis=0, keepdims=True)
        idx = jnp.argmin(st, axis=0, keepdims=True)
        onehot = iota == idx
        if k == 0:
            val0 = val
            w = jnp.ones_like(val)
            wsum = w
        else:
            w = jnp.exp((val0 - val) * (1.0 / BANDWIDTH))
            wsum = wsum + w
        w_mat = jnp.where(onehot, jnp.broadcast_to(w, st.shape), w_mat)
        if k < K - 1:
            st = jnp.where(onehot, jnp.float32(jnp.inf), st)

    w_mat = w_mat * (1.0 / wsum)
    y = jax.lax.dot_general(
        w_mat, bs_ref[...],
        dimension_numbers=(((0,), (0,)), ((), ())),
        preferred_element_type=jnp.float32,
    )
    y_ref[...] = y

    vs = jnp.sign(y)
    nv = jnp.maximum(jnp.sqrt(jnp.sum(y * y, axis=1, keepdims=True)), 1e-8)
    ns = jnp.maximum(jnp.sqrt(jnp.sum(vs * vs, axis=1, keepdims=True)), 1e-8)
    cos = jnp.sum(y * vs, axis=1, keepdims=True) / (nv * ns)
    blocksum = jnp.sum(1.0 - cos).reshape(1, 1)

    i = pl.program_id(0)

    @pl.when(i == 0)
    def _init():
        q_ref[...] = jnp.zeros_like(q_ref)

    q_ref[...] += blocksum

    @pl.when(i == nsteps - 1)
    def _fin():
        q_ref[...] = q_ref[...] * (1.0 / N)


@jax.jit
def kernel(x, centroids, base_set):
    ct = jnp.zeros((D, MP), jnp.float32).at[:, :M].set(centroids.T)
    bias = jnp.zeros((1, MP), jnp.float32).at[0, M:].set(3e38)
    bs = jnp.zeros((MP, NBIT), jnp.float32).at[:M, :].set(base_set)

    nsteps = N // BLOCK
    y, q = pl.pallas_call(
        functools.partial(_body, nsteps=nsteps),
        grid=(nsteps,),
        in_specs=[
            pl.BlockSpec((BLOCK, D), lambda i: (i, 0)),
            pl.BlockSpec((D, MP), lambda i: (0, 0)),
            pl.BlockSpec((1, MP), lambda i: (0, 0)),
            pl.BlockSpec((MP, NBIT), lambda i: (0, 0)),
        ],
        out_specs=[
            pl.BlockSpec((BLOCK, NBIT), lambda i: (i, 0)),
            pl.BlockSpec((1, 1), lambda i: (0, 0)),
        ],
        out_shape=[
            jax.ShapeDtypeStruct((N, NBIT), jnp.float32),
            jax.ShapeDtypeStruct((1, 1), jnp.float32),
        ],
        compiler_params=pltpu.CompilerParams(
            dimension_semantics=("arbitrary",),
        ),
    )(x, ct, bias, bs)
    return y, q[0, 0]
